# baseline (speedup 1.0000x reference)
import functools

import jax
import jax.numpy as jnp
from jax import lax
from jax.experimental import pallas as pl
from jax.experimental.pallas import tpu as pltpu
from jax.experimental.pallas import tpu_sc as plsc

_FIELD_NUM = 26
_DIM = 30
_NUM_REGIONS = 5
_EPS = 1e-5

_NC, _NS = 2, 16
_NW = _NC * _NS
_CHUNK = 128


def _sc_gather(emb_table, ids_flat):
    n = ids_flat.shape[0]
    rows_per_w = n // _NW
    nchunks = rows_per_w // _CHUNK
    mesh = plsc.VectorSubcoreMesh(core_axis_name="c", subcore_axis_name="s")

    @functools.partial(
        pl.kernel,
        mesh=mesh,
        out_type=jax.ShapeDtypeStruct((n, _DIM), jnp.float32),
        compiler_params=pltpu.CompilerParams(use_tc_tiling_on_sc=False),
        scratch_types=[
            pltpu.VMEM((rows_per_w,), jnp.int32),
            pltpu.VMEM((rows_per_w, _DIM), jnp.float32),
            pltpu.SemaphoreType.DMA,
        ],
    )
    def k(table_hbm, ids_hbm, out_hbm, idx_v, rows_v, sem):
        wid = lax.axis_index("s") * _NC + lax.axis_index("c")
        base = wid * rows_per_w
        pltpu.sync_copy(ids_hbm.at[pl.ds(base, rows_per_w)], idx_v)
        copies = []
        for j in range(nchunks):
            copies.append(pltpu.async_copy(
                table_hbm.at[idx_v.at[pl.ds(j * _CHUNK, _CHUNK)]],
                rows_v.at[pl.ds(j * _CHUNK, _CHUNK)],
                sem,
            ))
        for c in copies:
            c.wait()
        pltpu.sync_copy(rows_v, out_hbm.at[pl.ds(base, rows_per_w)])

    return k(emb_table, ids_flat)


def _tc_normalize_body(x_ref, w_ref, o_ref):
    xx = x_ref[...]
    b = x_ref.shape[0]
    cdim = _FIELD_NUM * _DIM
    s = jnp.sum(xx, axis=0, keepdims=True)
    ss = jnp.sum(xx * xx, axis=0, keepdims=True)

    c_i = lax.broadcasted_iota(jnp.int32, (cdim, _FIELD_NUM), 0)
    f_i = lax.broadcasted_iota(jnp.int32, (cdim, _FIELD_NUM), 1)
    g = (c_i // _DIM == f_i).astype(jnp.float32)

    cnt = float(b * _DIM)
    sum_f = jnp.dot(s, g, preferred_element_type=jnp.float32)
    ss_f = jnp.dot(ss, g, preferred_element_type=jnp.float32)
    mean_f = sum_f / cnt
    var_f = ss_f / cnt - mean_f * mean_f
    rstd_f = lax.rsqrt(var_f + _EPS)
    mean_c = jnp.dot(mean_f, g.T, preferred_element_type=jnp.float32)
    rstd_c = jnp.dot(rstd_f, g.T, preferred_element_type=jnp.float32)

    wdim = _FIELD_NUM * _NUM_REGIONS
    e = jnp.exp(w_ref[...])
    q_i = lax.broadcasted_iota(jnp.int32, (wdim, _FIELD_NUM), 0)
    f2_i = lax.broadcasted_iota(jnp.int32, (wdim, _FIELD_NUM), 1)
    q = (q_i // _NUM_REGIONS == f2_i).astype(jnp.float32)
    den_f = jnp.dot(e, q, preferred_element_type=jnp.float32)
    den_c = jnp.dot(den_f, q.T, preferred_element_type=jnp.float32)
    w_n = e / den_c

    qq = lax.broadcasted_iota(jnp.int32, (wdim, cdim), 0)
    cc = lax.broadcasted_iota(jnp.int32, (wdim, cdim), 1)
    d = cc % _DIM
    rid = (1 + (d >= 2).astype(jnp.int32) + (d >= 6).astype(jnp.int32)
           + (d >= 14).astype(jnp.int32))
    kmat = ((qq // _NUM_REGIONS == cc // _DIM)
            & (qq % _NUM_REGIONS == rid)).astype(jnp.float32)
    scale_c = jnp.dot(w_n, kmat, preferred_element_type=jnp.float32)

    o_ref[...] = (xx - mean_c) * (rstd_c * scale_c)


def _tc_normalize(x, w_flat):
    return pl.pallas_call(
        _tc_normalize_body,
        out_shape=jax.ShapeDtypeStruct(x.shape, jnp.float32),
    )(x, w_flat)


def kernel(input_ids, emb_table, field_region_weights):
    b, f = input_ids.shape
    ids_flat = input_ids.reshape(-1)
    gathered = _sc_gather(emb_table, ids_flat)
    x = gathered.reshape(b, f * _DIM)
    w_flat = field_region_weights.reshape(1, f * _NUM_REGIONS)
    y = _tc_normalize(x, w_flat)
    return y.reshape(b, f, _DIM)


# speedup vs baseline: 2.0250x; 2.0250x over previous
import functools

import jax
import jax.numpy as jnp
from jax import lax
from jax.experimental import pallas as pl
from jax.experimental.pallas import tpu as pltpu
from jax.experimental.pallas import tpu_sc as plsc

_FIELD_NUM = 26
_DIM = 30
_PDIM = 32
_NUM_REGIONS = 5
_EPS = 1e-5

_NC, _NS = 2, 16
_NW = _NC * _NS
_CHUNK = 128

_RB = 4096
_PACK = 128 // _PDIM


_CB = 1024
_TAIL_BLKS = 4


def _repack_body(nb, in0, in1, in2, in3, tail, out_ref):
    def tp(r):
        t = jnp.transpose(r[...])
        return jnp.concatenate(
            [t, jnp.zeros((_CB, _PDIM - _DIM), jnp.float32)], axis=1)

    pid = pl.program_id(0)
    pieces = [tp(in0), tp(in1), tp(in2),
              jnp.where(pid >= nb - _TAIL_BLKS, tp(tail), tp(in3))]
    out_ref[...] = jnp.concatenate(pieces, axis=1)


def _repack(table_t, tail_t, quarter):
    nb = quarter // _CB
    last = table_t.shape[1] // _CB - 1
    specs = [
        pl.BlockSpec(
            (_DIM, _CB),
            lambda c, m=m: (0, jnp.minimum(c + m * nb, last)))
        for m in range(_PACK)
    ]
    specs.append(pl.BlockSpec(
        (_DIM, _CB),
        lambda c: (0, jnp.clip(c - (nb - _TAIL_BLKS), 0, _TAIL_BLKS - 1))))
    return pl.pallas_call(
        functools.partial(_repack_body, nb),
        grid=(nb,),
        in_specs=specs,
        out_specs=pl.BlockSpec((_CB, 128), lambda c: (c, 0)),
        out_shape=jax.ShapeDtypeStruct((quarter, 128), jnp.float32),
    )(table_t, table_t, table_t, table_t, tail_t)


def _sc_gather(xlin, ids_flat, quarter):
    n = ids_flat.shape[0]
    rows_per_w = n // _NW
    nchunks = rows_per_w // _CHUNK
    mesh = plsc.VectorSubcoreMesh(core_axis_name="c", subcore_axis_name="s")

    @functools.partial(
        pl.kernel,
        mesh=mesh,
        out_type=jax.ShapeDtypeStruct((n, _PDIM), jnp.float32),
        compiler_params=pltpu.CompilerParams(use_tc_tiling_on_sc=False),
        scratch_types=[
            pltpu.VMEM((rows_per_w,), jnp.int32),
            pltpu.VMEM((rows_per_w, _PDIM), jnp.float32),
            pltpu.SemaphoreType.DMA,
        ],
    )
    def k(table_hbm, ids_hbm, out_hbm, idx_v, rows_v, sem):
        wid = lax.axis_index("s") * _NC + lax.axis_index("c")
        base = wid * rows_per_w
        pltpu.sync_copy(ids_hbm.at[pl.ds(base, rows_per_w)], idx_v)

        copies = []
        for j in range(nchunks):
            copies.append(pltpu.async_copy(
                table_hbm.at[idx_v.at[pl.ds(j * _CHUNK, _CHUNK)]],
                rows_v.at[pl.ds(j * _CHUNK, _CHUNK)],
                sem,
            ))
        for c in copies:
            c.wait()
        pltpu.sync_copy(rows_v, out_hbm.at[pl.ds(base, rows_per_w)])

    return k(xlin, ids_flat)


def _tc_normalize_body(x_ref, w_ref, o_ref):
    xx = x_ref[...]
    b = x_ref.shape[0]
    cdim = _FIELD_NUM * _DIM
    s = jnp.sum(xx, axis=0, keepdims=True)
    ss = jnp.sum(xx * xx, axis=0, keepdims=True)

    c_i = lax.broadcasted_iota(jnp.int32, (cdim, _FIELD_NUM), 0)
    f_i = lax.broadcasted_iota(jnp.int32, (cdim, _FIELD_NUM), 1)
    g = (c_i // _DIM == f_i).astype(jnp.float32)

    cnt = float(b * _DIM)
    sum_f = jnp.dot(s, g, preferred_element_type=jnp.float32,
                    precision=lax.Precision.HIGHEST)
    ss_f = jnp.dot(ss, g, preferred_element_type=jnp.float32,
                    precision=lax.Precision.HIGHEST)
    mean_f = sum_f / cnt
    var_f = ss_f / cnt - mean_f * mean_f
    rstd_f = lax.rsqrt(var_f + _EPS)
    mean_c = jnp.dot(mean_f, g.T, preferred_element_type=jnp.float32,
                    precision=lax.Precision.HIGHEST)
    rstd_c = jnp.dot(rstd_f, g.T, preferred_element_type=jnp.float32,
                    precision=lax.Precision.HIGHEST)

    wdim = _FIELD_NUM * _NUM_REGIONS
    e = jnp.exp(w_ref[...])
    q_i = lax.broadcasted_iota(jnp.int32, (wdim, _FIELD_NUM), 0)
    f2_i = lax.broadcasted_iota(jnp.int32, (wdim, _FIELD_NUM), 1)
    q = (q_i // _NUM_REGIONS == f2_i).astype(jnp.float32)
    den_f = jnp.dot(e, q, preferred_element_type=jnp.float32,
                    precision=lax.Precision.HIGHEST)
    den_c = jnp.dot(den_f, q.T, preferred_element_type=jnp.float32,
                    precision=lax.Precision.HIGHEST)
    w_n = e / den_c

    qq = lax.broadcasted_iota(jnp.int32, (wdim, cdim), 0)
    cc = lax.broadcasted_iota(jnp.int32, (wdim, cdim), 1)
    d = cc % _DIM
    rid = (1 + (d >= 2).astype(jnp.int32) + (d >= 6).astype(jnp.int32)
           + (d >= 14).astype(jnp.int32))
    kmat = ((qq // _NUM_REGIONS == cc // _DIM)
            & (qq % _NUM_REGIONS == rid)).astype(jnp.float32)
    scale_c = jnp.dot(w_n, kmat, preferred_element_type=jnp.float32,
                    precision=lax.Precision.HIGHEST)

    o_ref[...] = (xx - mean_c) * (rstd_c * scale_c)


def _tc_normalize(x, w_flat):
    return pl.pallas_call(
        _tc_normalize_body,
        out_shape=jax.ShapeDtypeStruct(x.shape, jnp.float32),
    )(x, w_flat)


def kernel(input_ids, emb_table, field_region_weights):
    b, f = input_ids.shape
    v = emb_table.shape[0]
    quarter = ((v + _PACK * _CB - 1) // (_PACK * _CB)) * _CB
    table_t = emb_table.T
    tail_start = _PACK * quarter - _TAIL_BLKS * _CB
    tail_t = jnp.pad(
        table_t[:, tail_start:],
        ((0, 0), (0, _TAIL_BLKS * _CB - (v - tail_start))))
    packed = _repack(table_t, tail_t, quarter)
    xlin = packed.reshape(quarter * _PACK, _PDIM)
    ids_flat = input_ids.reshape(-1)
    ids_remap = (ids_flat % quarter) * _PACK + ids_flat // quarter
    gathered = _sc_gather(xlin, ids_remap, quarter)
    x = gathered[:, :_DIM].reshape(b, f * _DIM)
    w_flat = field_region_weights.reshape(1, f * _NUM_REGIONS)
    y = _tc_normalize(x, w_flat)
    return y.reshape(b, f, _DIM)


# speedup vs baseline: 3.0440x; 1.5032x over previous
import functools

import jax
import jax.numpy as jnp
from jax import lax
from jax.experimental import pallas as pl
from jax.experimental.pallas import tpu as pltpu
from jax.experimental.pallas import tpu_sc as plsc

_FIELD_NUM = 26
_DIM = 30
_PDIM = 32
_NUM_REGIONS = 5
_EPS = 1e-5

_NC, _NS = 2, 16
_NW = _NC * _NS
_CHUNK = 128

_RB = 4096
_PACK = 128 // _PDIM


_CB = 2048


def _repack_body(in0, in1, in2, in3, out_ref):
    z = jnp.zeros((_PDIM - _DIM, _CB), jnp.float32)
    x4 = jnp.concatenate(
        [in0[...], z, in1[...], z, in2[...], z, in3[...], z], axis=0)
    out_ref[...] = jnp.transpose(x4)


def _repack(table_t, p3, quarter):
    nb = quarter // _CB
    specs = [
        pl.BlockSpec((_DIM, _CB), lambda c, m=m: (0, c + m * nb))
        for m in range(_PACK - 1)
    ]
    specs.append(pl.BlockSpec((_DIM, _CB), lambda c: (0, c)))
    return pl.pallas_call(
        _repack_body,
        grid=(nb,),
        in_specs=specs,
        out_specs=pl.BlockSpec((_CB, 128), lambda c: (c, 0)),
        out_shape=jax.ShapeDtypeStruct((quarter, 128), jnp.float32),
    )(table_t, table_t, table_t, p3)


def _sc_gather(xlin, ids_flat, quarter):
    n = ids_flat.shape[0]
    rows_per_w = n // _NW
    nchunks = rows_per_w // _CHUNK
    mesh = plsc.VectorSubcoreMesh(core_axis_name="c", subcore_axis_name="s")

    @functools.partial(
        pl.kernel,
        mesh=mesh,
        out_type=jax.ShapeDtypeStruct((n, _PDIM), jnp.float32),
        compiler_params=pltpu.CompilerParams(use_tc_tiling_on_sc=False),
        scratch_types=[
            pltpu.VMEM((rows_per_w,), jnp.int32),
            pltpu.VMEM((rows_per_w, _PDIM), jnp.float32),
            pltpu.SemaphoreType.DMA,
        ],
    )
    def k(table_hbm, ids_hbm, out_hbm, idx_v, rows_v, sem):
        wid = lax.axis_index("s") * _NC + lax.axis_index("c")
        base = wid * rows_per_w
        pltpu.sync_copy(ids_hbm.at[pl.ds(base, rows_per_w)], idx_v)

        copies = []
        for j in range(nchunks):
            copies.append(pltpu.async_copy(
                table_hbm.at[idx_v.at[pl.ds(j * _CHUNK, _CHUNK)]],
                rows_v.at[pl.ds(j * _CHUNK, _CHUNK)],
                sem,
            ))
        for c in copies:
            c.wait()
        pltpu.sync_copy(rows_v, out_hbm.at[pl.ds(base, rows_per_w)])

    return k(xlin, ids_flat)


def _tc_normalize_body(x_ref, w_ref, o_ref):
    xx = x_ref[...]
    b = x_ref.shape[0]
    cdim = _FIELD_NUM * _DIM
    s = jnp.sum(xx, axis=0, keepdims=True)
    ss = jnp.sum(xx * xx, axis=0, keepdims=True)

    c_i = lax.broadcasted_iota(jnp.int32, (cdim, _FIELD_NUM), 0)
    f_i = lax.broadcasted_iota(jnp.int32, (cdim, _FIELD_NUM), 1)
    g = (c_i // _DIM == f_i).astype(jnp.float32)

    cnt = float(b * _DIM)
    sum_f = jnp.dot(s, g, preferred_element_type=jnp.float32,
                    precision=lax.Precision.HIGHEST)
    ss_f = jnp.dot(ss, g, preferred_element_type=jnp.float32,
                    precision=lax.Precision.HIGHEST)
    mean_f = sum_f / cnt
    var_f = ss_f / cnt - mean_f * mean_f
    rstd_f = lax.rsqrt(var_f + _EPS)
    mean_c = jnp.dot(mean_f, g.T, preferred_element_type=jnp.float32,
                    precision=lax.Precision.HIGHEST)
    rstd_c = jnp.dot(rstd_f, g.T, preferred_element_type=jnp.float32,
                    precision=lax.Precision.HIGHEST)

    wdim = _FIELD_NUM * _NUM_REGIONS
    e = jnp.exp(w_ref[...])
    q_i = lax.broadcasted_iota(jnp.int32, (wdim, _FIELD_NUM), 0)
    f2_i = lax.broadcasted_iota(jnp.int32, (wdim, _FIELD_NUM), 1)
    q = (q_i // _NUM_REGIONS == f2_i).astype(jnp.float32)
    den_f = jnp.dot(e, q, preferred_element_type=jnp.float32,
                    precision=lax.Precision.HIGHEST)
    den_c = jnp.dot(den_f, q.T, preferred_element_type=jnp.float32,
                    precision=lax.Precision.HIGHEST)
    w_n = e / den_c

    qq = lax.broadcasted_iota(jnp.int32, (wdim, cdim), 0)
    cc = lax.broadcasted_iota(jnp.int32, (wdim, cdim), 1)
    d = cc % _DIM
    rid = (1 + (d >= 2).astype(jnp.int32) + (d >= 6).astype(jnp.int32)
           + (d >= 14).astype(jnp.int32))
    kmat = ((qq // _NUM_REGIONS == cc // _DIM)
            & (qq % _NUM_REGIONS == rid)).astype(jnp.float32)
    scale_c = jnp.dot(w_n, kmat, preferred_element_type=jnp.float32,
                    precision=lax.Precision.HIGHEST)

    o_ref[...] = (xx - mean_c) * (rstd_c * scale_c)


def _tc_normalize(x, w_flat):
    return pl.pallas_call(
        _tc_normalize_body,
        out_shape=jax.ShapeDtypeStruct(x.shape, jnp.float32),
    )(x, w_flat)


def kernel(input_ids, emb_table, field_region_weights):
    b, f = input_ids.shape
    v = emb_table.shape[0]
    quarter = ((v + _PACK * _CB - 1) // (_PACK * _CB)) * _CB
    table_t = emb_table.T
    p3 = jnp.pad(table_t[:, (_PACK - 1) * quarter:],
                 ((0, 0), (0, _PACK * quarter - v)))
    packed = _repack(table_t, p3, quarter)
    xlin = packed.reshape(quarter * _PACK, _PDIM)
    ids_flat = input_ids.reshape(-1)
    ids_remap = (ids_flat % quarter) * _PACK + ids_flat // quarter
    gathered = _sc_gather(xlin, ids_remap, quarter)
    x = gathered[:, :_DIM].reshape(b, f * _DIM)
    w_flat = field_region_weights.reshape(1, f * _NUM_REGIONS)
    y = _tc_normalize(x, w_flat)
    return y.reshape(b, f, _DIM)
